# Initial kernel scaffold; baseline (speedup 1.0000x reference)
import jax
import jax.numpy as jnp
from jax.experimental import pallas as pl


def kernel(x, edge_index, W1, att_src1, att_dst1, b1, W2, att_src2, att_dst2, b2):
    raise NotImplementedError("write your pallas kernel here")


# speedup vs baseline: 54.8309x; 54.8309x over previous
import functools

import jax
import jax.numpy as jnp
from jax import lax
from jax.experimental import pallas as pl
from jax.experimental.pallas import tpu as pltpu
from jax.experimental.pallas import tpu_sc as plsc

N = 100000
E = 3200000
NC = 2
NS = 16
CHUNK = 128
CPT = -(-E // (NS * CHUNK))
EPT = CPT * CHUNK
EP = EPT * NS
SENT = 2 * N
NP = 100096
RPT = NP // NS


def _edge_body(H, sidx_hbm, didx_hbm, tab_hbm, dtab_hbm, mb_hbm, out_hbm,
               sidx_v, didx_v, sidxc_v, didxc_v, srows, drows, stage, mvec_v,
               acc):
    c_ax = lax.axis_index("c")
    s_ax = lax.axis_index("s")
    io = lax.iota(jnp.int32, 16)

    zero = jnp.zeros((16,), jnp.float32)
    for e in range(CHUNK):
        stage[e, pl.ds(0, 16)] = zero
    lo = s_ax * RPT
    nfull = RPT // CHUNK
    for r in range(nfull):
        pltpu.sync_copy(stage, acc.at[pl.ds(lo + r * CHUNK, CHUNK)])
    rem = RPT - nfull * CHUNK
    if rem:
        pltpu.sync_copy(stage.at[pl.ds(0, rem)],
                        acc.at[pl.ds(lo + nfull * CHUNK, rem)])
    pltpu.sync_copy(mb_hbm.at[c_ax], mvec_v)
    plsc.subcore_barrier()

    mv = mvec_v[pl.ds(0, 16)]
    if H == 4:
        pat = jnp.where(io < 8, io >> 2, jnp.where(io == 9, 1, 0)) + 10
    else:
        pat = jnp.full((16,), 10, jnp.int32)
    gdn = lax.GatherDimensionNumbers(
        offset_dims=(), collapsed_slice_dims=(0,), start_index_map=(0,))
    coff = c_ax * N
    ebase = s_ax * EPT

    def chunk(c, carry):
        eb = ebase + c * CHUNK
        pltpu.sync_copy(sidx_hbm.at[pl.ds(eb, CHUNK)], sidx_v)
        pltpu.sync_copy(didx_hbm.at[pl.ds(eb, CHUNK)], didx_v)
        for g in range(CHUNK // 16):
            sl = pl.ds(g * 16, 16)
            sidxc_v[sl] = jnp.minimum(sidx_v[sl] + coff, SENT)
            didxc_v[sl] = jnp.minimum(didx_v[sl] + coff, SENT)
        pltpu.sync_copy(tab_hbm.at[sidxc_v], srows)
        pltpu.sync_copy(dtab_hbm.at[didxc_v], drows)
        for e in range(CHUNK):
            rowv = srows[e, pl.ds(0, 16)]
            drowv = drows[e, pl.ds(0, 16)]
            t = rowv + drowv
            lrt = jnp.maximum(t, t * jnp.float32(0.2))
            wv = jnp.exp(lrt - mv)
            wpat = lax.gather(wv, pat[:, None], gdn, slice_sizes=(1,),
                              mode=lax.GatherScatterMode.PROMISE_IN_BOUNDS)
            stage[e, pl.ds(0, 16)] = rowv * wpat
        pltpu.sync_copy(stage, acc.at[didx_v], add=True)
        return carry

    lax.fori_loop(0, CPT, chunk, 0)
    plsc.subcore_barrier()
    pltpu.sync_copy(acc.at[pl.ds(lo, RPT)], out_hbm.at[c_ax, pl.ds(lo, RPT)])


def _make_edge_pass(H):
    mesh = plsc.VectorSubcoreMesh(core_axis_name="c", subcore_axis_name="s")
    return pl.kernel(
        functools.partial(_edge_body, H),
        mesh=mesh,
        compiler_params=pltpu.CompilerParams(use_tc_tiling_on_sc=False),
        out_type=jax.ShapeDtypeStruct((NC, NP, 16), jnp.float32),
        scratch_types=[
            pltpu.VMEM((CHUNK,), jnp.int32),
            pltpu.VMEM((CHUNK,), jnp.int32),
            pltpu.VMEM((CHUNK,), jnp.int32),
            pltpu.VMEM((CHUNK,), jnp.int32),
            pltpu.VMEM((CHUNK, 16), jnp.float32),
            pltpu.VMEM((CHUNK, 16), jnp.float32),
            pltpu.VMEM((CHUNK, 16), jnp.float32),
            pltpu.VMEM((16,), jnp.float32),
            pltpu.VMEM_SHARED((NP, 16), jnp.float32),
        ],
    )


_edge_pass = {4: _make_edge_pass(4), 1: _make_edge_pass(1)}


def _leaky(t):
    return jnp.maximum(t, 0.2 * t)


def _gat_layer(x, srcp, dstp, W, att_src, att_dst, heads, out_ch):
    h = x @ W
    hr = h.reshape(N, heads, out_ch)
    a_src = (hr * att_src).sum(-1)
    a_dst = (hr * att_dst).sum(-1)
    m = _leaky(a_src.max(0) + a_dst.max(0))

    one = jnp.ones((N, 1), jnp.float32)
    z4 = jnp.zeros((N, 4), jnp.float32)
    z10 = jnp.zeros((N, 10), jnp.float32)
    if heads == 4:
        rows = [jnp.concatenate(
            [h[:, 8 * c:8 * c + 8], one, one,
             a_src[:, 2 * c:2 * c + 1], a_src[:, 2 * c + 1:2 * c + 2], z4], 1)
            for c in range(2)]
        drows = [jnp.concatenate(
            [z10, a_dst[:, 2 * c:2 * c + 1], a_dst[:, 2 * c + 1:2 * c + 2],
             z4], 1) for c in range(2)]
        mrows = [jnp.concatenate(
            [jnp.full((10,), 60.0, jnp.float32), m[2 * c:2 * c + 1],
             m[2 * c + 1:2 * c + 2], jnp.full((4,), 60.0, jnp.float32)])
            for c in range(2)]
    else:
        rows = [jnp.concatenate(
            [h[:, 8 * c:8 * c + 8], one, one, a_src, a_src, z4], 1)
            for c in range(2)]
        drows = [jnp.concatenate([z10, a_dst, a_dst, z4], 1)
                 for _ in range(2)]
        mrows = [jnp.concatenate(
            [jnp.full((10,), 60.0, jnp.float32), m, m,
             jnp.full((4,), 60.0, jnp.float32)]) for _ in range(2)]

    sentinel = jnp.zeros((8, 16), jnp.float32)
    stab = jnp.concatenate([rows[0], rows[1], sentinel], 0)
    dtab = jnp.concatenate([drows[0], drows[1], sentinel], 0)
    mb = jnp.stack(mrows)

    acc = _edge_pass[heads](srcp, dstp, stab, dtab, mb)
    ws = jnp.exp(_leaky(a_src + a_dst) - m[None, :])
    num = jnp.concatenate([acc[0, :N, 0:8], acc[1, :N, 0:8]], 1)
    num = num + (hr * ws[..., None]).reshape(N, 16)
    if heads == 4:
        den = jnp.concatenate([acc[0, :N, 8:10], acc[1, :N, 8:10]], 1) + ws
    else:
        den = acc[0, :N, 8:9] + ws
    return num / (jnp.repeat(den, out_ch, axis=1) + 1e-16)


def kernel(x, edge_index, W1, att_src1, att_dst1, b1, W2, att_src2, att_dst2,
           b2):
    src = edge_index[0].astype(jnp.int32)
    dst = edge_index[1].astype(jnp.int32)
    pad = EP - E
    srcp = jnp.concatenate([src, jnp.full((pad,), SENT, jnp.int32)])
    dstp = jnp.concatenate([dst, jnp.zeros((pad,), jnp.int32)])

    g1 = _gat_layer(x, srcp, dstp, W1, att_src1, att_dst1, 4, 4) + b1
    x2 = jax.nn.elu(g1)
    g2 = _gat_layer(x2, srcp, dstp, W2, att_src2, att_dst2, 1, 16) + b2
    return g2


# speedup vs baseline: 152.4272x; 2.7799x over previous
import functools

import jax
import jax.numpy as jnp
from jax import lax
from jax.experimental import pallas as pl
from jax.experimental.pallas import tpu as pltpu
from jax.experimental.pallas import tpu_sc as plsc

N = 100000
E = 3200000
NC = 2
NS = 16
CHUNK = 128
NB = 4
CPT = -(-E // (NS * CHUNK * NB)) * NB
EPT = CPT * CHUNK
EP = EPT * NS
SENT = 2 * N
NP = 100096
RPT = NP // NS


def _edge_body(H, sidx_hbm, didx_hbm, tab_hbm, dtab_hbm, mb_hbm, out_hbm,
               *scr):
    SIDX, DIDX, SIDXC, DIDXC, DIDXS = (scr[i * NB:(i + 1) * NB]
                                       for i in range(5))
    SROWS = scr[5 * NB:6 * NB]
    DROWS = scr[6 * NB:7 * NB]
    STAGE = scr[7 * NB:8 * NB]
    mvec_v = scr[8 * NB]
    acc = scr[8 * NB + 1]
    SEMI = scr[8 * NB + 2:8 * NB + 2 + NB]
    SEMG = scr[8 * NB + 2 + NB:8 * NB + 2 + 2 * NB]
    SEMS = scr[8 * NB + 2 + 2 * NB:8 * NB + 2 + 3 * NB]

    c_ax = lax.axis_index("c")
    s_ax = lax.axis_index("s")
    io = lax.iota(jnp.int32, 16)

    zero = jnp.zeros((16,), jnp.float32)
    zi = jnp.zeros((16,), jnp.int32)
    for b in range(NB):
        for e in range(CHUNK):
            STAGE[b][e, pl.ds(0, 16)] = zero
        for g in range(CHUNK // 16):
            DIDXS[b][pl.ds(g * 16, 16)] = zi
    lo = s_ax * RPT
    nfull = RPT // CHUNK
    for r in range(nfull):
        pltpu.sync_copy(STAGE[0], acc.at[pl.ds(lo + r * CHUNK, CHUNK)])
    rem = RPT - nfull * CHUNK
    if rem:
        pltpu.sync_copy(STAGE[0].at[pl.ds(0, rem)],
                        acc.at[pl.ds(lo + nfull * CHUNK, rem)])
    pltpu.sync_copy(mb_hbm.at[c_ax], mvec_v)
    plsc.subcore_barrier()

    mv = mvec_v[pl.ds(0, 16)]
    if H == 4:
        pat = jnp.where(io < 8, io >> 2, jnp.where(io == 9, 1, 0)) + 10
    else:
        pat = jnp.full((16,), 10, jnp.int32)
    gdn = lax.GatherDimensionNumbers(
        offset_dims=(), collapsed_slice_dims=(0,), start_index_map=(0,))
    coff = c_ax * N
    ebase = s_ax * EPT

    def issue_idx(c, b):
        eb = ebase + c * CHUNK
        pltpu.async_copy(sidx_hbm.at[pl.ds(eb, CHUNK)], SIDX[b], SEMI[b])
        pltpu.async_copy(didx_hbm.at[pl.ds(eb, CHUNK)], DIDX[b], SEMI[b])

    def wait_idx(b):
        pltpu.make_async_copy(sidx_hbm.at[pl.ds(0, CHUNK)], SIDX[b],
                              SEMI[b]).wait()
        pltpu.make_async_copy(didx_hbm.at[pl.ds(0, CHUNK)], DIDX[b],
                              SEMI[b]).wait()

    def comp_idxc(b):
        for g in range(CHUNK // 16):
            sl = pl.ds(g * 16, 16)
            DIDXS[b][sl] = DIDX[b][sl]
            SIDXC[b][sl] = jnp.minimum(SIDX[b][sl] + coff, SENT)
            DIDXC[b][sl] = jnp.minimum(DIDX[b][sl] + coff, SENT)

    def issue_gather(b):
        pltpu.async_copy(tab_hbm.at[SIDXC[b]], SROWS[b], SEMG[b])
        pltpu.async_copy(dtab_hbm.at[DIDXC[b]], DROWS[b], SEMG[b])

    def wait_gather(b):
        pltpu.make_async_copy(tab_hbm.at[SIDXC[b]], SROWS[b], SEMG[b]).wait()
        pltpu.make_async_copy(dtab_hbm.at[DIDXC[b]], DROWS[b], SEMG[b]).wait()

    def issue_scatter(b):
        pltpu.async_copy(STAGE[b], acc.at[DIDXS[b]], SEMS[b], add=True)

    def wait_scatter(b):
        pltpu.make_async_copy(STAGE[b], acc.at[DIDXS[b]], SEMS[b]).wait()

    def compute(b):
        for e in range(CHUNK):
            rowv = SROWS[b][e, pl.ds(0, 16)]
            drowv = DROWS[b][e, pl.ds(0, 16)]
            t = rowv + drowv
            lrt = jnp.maximum(t, t * jnp.float32(0.2))
            wv = jnp.exp(lrt - mv)
            wpat = lax.gather(wv, pat[:, None], gdn, slice_sizes=(1,),
                              mode=lax.GatherScatterMode.PROMISE_IN_BOUNDS)
            STAGE[b][e, pl.ds(0, 16)] = rowv * wpat

    issue_idx(0, 0)
    issue_idx(1, 1)
    issue_idx(2, 2)
    wait_idx(0)
    comp_idxc(0)
    issue_gather(0)
    wait_idx(1)
    comp_idxc(1)
    issue_gather(1)
    issue_scatter(2)
    issue_scatter(3)

    def step(c4, carry):
        for j in range(NB):
            c = c4 * NB + j
            bq = (j + 2) % NB
            wait_scatter(bq)

            @pl.when(c + 2 < CPT)
            def _():
                wait_idx(bq)
                comp_idxc(bq)
                issue_gather(bq)

            @pl.when(c + 3 < CPT)
            def _():
                issue_idx(c + 3, (j + 3) % NB)

            wait_gather(j)
            compute(j)
            issue_scatter(j)
        return carry

    lax.fori_loop(0, CPT // NB, step, 0)
    wait_scatter(2)
    wait_scatter(3)
    plsc.subcore_barrier()
    pltpu.sync_copy(acc.at[pl.ds(lo, RPT)], out_hbm.at[c_ax, pl.ds(lo, RPT)])


def _make_edge_pass(H):
    mesh = plsc.VectorSubcoreMesh(core_axis_name="c", subcore_axis_name="s")
    return pl.kernel(
        functools.partial(_edge_body, H),
        mesh=mesh,
        compiler_params=pltpu.CompilerParams(use_tc_tiling_on_sc=False),
        out_type=jax.ShapeDtypeStruct((NC, NP, 16), jnp.float32),
        scratch_types=(
            [pltpu.VMEM((CHUNK,), jnp.int32)] * (5 * NB)
            + [pltpu.VMEM((CHUNK, 16), jnp.float32)] * (3 * NB)
            + [pltpu.VMEM((16,), jnp.float32)]
            + [pltpu.VMEM_SHARED((NP, 16), jnp.float32)]
            + [pltpu.SemaphoreType.DMA] * (3 * NB)
        ),
    )


_edge_pass = {4: _make_edge_pass(4), 1: _make_edge_pass(1)}


def _leaky(t):
    return jnp.maximum(t, 0.2 * t)


def _gat_layer(x, srcp, dstp, W, att_src, att_dst, heads, out_ch):
    h = x @ W
    hr = h.reshape(N, heads, out_ch)
    a_src = (hr * att_src).sum(-1)
    a_dst = (hr * att_dst).sum(-1)
    m = _leaky(a_src.max(0) + a_dst.max(0))

    one = jnp.ones((N, 1), jnp.float32)
    z4 = jnp.zeros((N, 4), jnp.float32)
    z10 = jnp.zeros((N, 10), jnp.float32)
    if heads == 4:
        rows = [jnp.concatenate(
            [h[:, 8 * c:8 * c + 8], one, one,
             a_src[:, 2 * c:2 * c + 1], a_src[:, 2 * c + 1:2 * c + 2], z4], 1)
            for c in range(2)]
        drows = [jnp.concatenate(
            [z10, a_dst[:, 2 * c:2 * c + 1], a_dst[:, 2 * c + 1:2 * c + 2],
             z4], 1) for c in range(2)]
        mrows = [jnp.concatenate(
            [jnp.full((10,), 60.0, jnp.float32), m[2 * c:2 * c + 1],
             m[2 * c + 1:2 * c + 2], jnp.full((4,), 60.0, jnp.float32)])
            for c in range(2)]
    else:
        rows = [jnp.concatenate(
            [h[:, 8 * c:8 * c + 8], one, one, a_src, a_src, z4], 1)
            for c in range(2)]
        drows = [jnp.concatenate([z10, a_dst, a_dst, z4], 1)
                 for _ in range(2)]
        mrows = [jnp.concatenate(
            [jnp.full((10,), 60.0, jnp.float32), m, m,
             jnp.full((4,), 60.0, jnp.float32)]) for _ in range(2)]

    sentinel = jnp.zeros((8, 16), jnp.float32)
    stab = jnp.concatenate([rows[0], rows[1], sentinel], 0)
    dtab = jnp.concatenate([drows[0], drows[1], sentinel], 0)
    mb = jnp.stack(mrows)

    acc = _edge_pass[heads](srcp, dstp, stab, dtab, mb)
    ws = jnp.exp(_leaky(a_src + a_dst) - m[None, :])
    num = jnp.concatenate([acc[0, :N, 0:8], acc[1, :N, 0:8]], 1)
    num = num + (hr * ws[..., None]).reshape(N, 16)
    if heads == 4:
        den = jnp.concatenate([acc[0, :N, 8:10], acc[1, :N, 8:10]], 1) + ws
    else:
        den = acc[0, :N, 8:9] + ws
    return num / (jnp.repeat(den, out_ch, axis=1) + 1e-16)


def kernel(x, edge_index, W1, att_src1, att_dst1, b1, W2, att_src2, att_dst2,
           b2):
    src = edge_index[0].astype(jnp.int32)
    dst = edge_index[1].astype(jnp.int32)
    pad = EP - E
    srcp = jnp.concatenate([src, jnp.full((pad,), SENT, jnp.int32)])
    dstp = jnp.concatenate([dst, jnp.zeros((pad,), jnp.int32)])

    g1 = _gat_layer(x, srcp, dstp, W1, att_src1, att_dst1, 4, 4) + b1
    x2 = jax.nn.elu(g1)
    g2 = _gat_layer(x2, srcp, dstp, W2, att_src2, att_dst2, 1, 16) + b2
    return g2


# speedup vs baseline: 160.9651x; 1.0560x over previous
import functools

import jax
import jax.numpy as jnp
from jax import lax
from jax.experimental import pallas as pl
from jax.experimental.pallas import tpu as pltpu
from jax.experimental.pallas import tpu_sc as plsc

N = 100000
E = 3200000
NC = 2
NS = 16
CHUNK = 128
NB = 4
CPT = -(-E // (NS * CHUNK * NB)) * NB
EPT = CPT * CHUNK
EP = EPT * NS
SENT = 2 * N
NP = 100096
RPT = NP // NS


def _edge_body(H, sidx_hbm, didx_hbm, tab_hbm, dtab_hbm, out_hbm, *scr):
    SIDX, DIDX, SIDXC, DIDXC, DIDXS = (scr[i * NB:(i + 1) * NB]
                                       for i in range(5))
    SROWS = scr[5 * NB:6 * NB]
    DROWS = scr[6 * NB:7 * NB]
    STAGE = scr[7 * NB:8 * NB]
    acc = scr[8 * NB]
    SEMI = scr[8 * NB + 1:8 * NB + 1 + NB]
    SEMG = scr[8 * NB + 1 + NB:8 * NB + 1 + 2 * NB]
    SEMS = scr[8 * NB + 1 + 2 * NB:8 * NB + 1 + 3 * NB]

    c_ax = lax.axis_index("c")
    s_ax = lax.axis_index("s")
    io = lax.iota(jnp.int32, 16)

    zero = jnp.zeros((16,), jnp.float32)
    zi = jnp.zeros((16,), jnp.int32)
    for b in range(NB):
        for e in range(CHUNK):
            STAGE[b][e, pl.ds(0, 16)] = zero
        for g in range(CHUNK // 16):
            DIDXS[b][pl.ds(g * 16, 16)] = zi
    lo = s_ax * RPT
    nfull = RPT // CHUNK
    for r in range(nfull):
        pltpu.sync_copy(STAGE[0], acc.at[pl.ds(lo + r * CHUNK, CHUNK)])
    rem = RPT - nfull * CHUNK
    if rem:
        pltpu.sync_copy(STAGE[0].at[pl.ds(0, rem)],
                        acc.at[pl.ds(lo + nfull * CHUNK, rem)])
    plsc.subcore_barrier()

    if H == 4:
        pat = jnp.where(io < 8, io >> 2, jnp.where(io == 9, 1, 0)) + 10
    else:
        pat = jnp.full((16,), 10, jnp.int32)
    gdn = lax.GatherDimensionNumbers(
        offset_dims=(), collapsed_slice_dims=(0,), start_index_map=(0,))
    coff = c_ax * N
    ebase = s_ax * EPT

    def issue_idx(c, b):
        eb = ebase + c * CHUNK
        pltpu.async_copy(sidx_hbm.at[pl.ds(eb, CHUNK)], SIDX[b], SEMI[b])
        pltpu.async_copy(didx_hbm.at[pl.ds(eb, CHUNK)], DIDX[b], SEMI[b])

    def wait_idx(b):
        pltpu.make_async_copy(sidx_hbm.at[pl.ds(0, CHUNK)], SIDX[b],
                              SEMI[b]).wait()
        pltpu.make_async_copy(didx_hbm.at[pl.ds(0, CHUNK)], DIDX[b],
                              SEMI[b]).wait()

    def comp_idxc(b):
        for g in range(CHUNK // 16):
            sl = pl.ds(g * 16, 16)
            DIDXS[b][sl] = DIDX[b][sl]
            SIDXC[b][sl] = jnp.minimum(SIDX[b][sl] + coff, SENT)
            DIDXC[b][sl] = jnp.minimum(DIDX[b][sl] + coff, SENT)

    def issue_gather(b):
        pltpu.async_copy(tab_hbm.at[SIDXC[b]], SROWS[b], SEMG[b])
        pltpu.async_copy(dtab_hbm.at[DIDXC[b]], DROWS[b], SEMG[b])

    def wait_gather(b):
        pltpu.make_async_copy(tab_hbm.at[SIDXC[b]], SROWS[b], SEMG[b]).wait()
        pltpu.make_async_copy(dtab_hbm.at[DIDXC[b]], DROWS[b], SEMG[b]).wait()

    def issue_scatter(b):
        pltpu.async_copy(STAGE[b], acc.at[DIDXS[b]], SEMS[b], add=True)

    def wait_scatter(b):
        pltpu.make_async_copy(STAGE[b], acc.at[DIDXS[b]], SEMS[b]).wait()

    def compute(b):
        for e in range(CHUNK):
            rowv = SROWS[b][e, pl.ds(0, 16)]
            drowv = DROWS[b][e, pl.ds(0, 16)]
            t = rowv + drowv
            wv = jnp.exp(jnp.maximum(t, t * jnp.float32(0.2)))
            wpat = lax.gather(wv, pat[:, None], gdn, slice_sizes=(1,),
                              mode=lax.GatherScatterMode.PROMISE_IN_BOUNDS)
            STAGE[b][e, pl.ds(0, 16)] = rowv * wpat

    issue_idx(0, 0)
    issue_idx(1, 1)
    issue_idx(2, 2)
    wait_idx(0)
    comp_idxc(0)
    issue_gather(0)
    wait_idx(1)
    comp_idxc(1)
    issue_gather(1)
    issue_scatter(2)
    issue_scatter(3)

    def step(c4, carry):
        for j in range(NB):
            c = c4 * NB + j
            bq = (j + 2) % NB
            wait_scatter(bq)

            @pl.when(c + 2 < CPT)
            def _():
                wait_idx(bq)
                comp_idxc(bq)
                issue_gather(bq)

            @pl.when(c + 3 < CPT)
            def _():
                issue_idx(c + 3, (j + 3) % NB)

            wait_gather(j)
            compute(j)
            issue_scatter(j)
        return carry

    lax.fori_loop(0, CPT // NB, step, 0)
    wait_scatter(2)
    wait_scatter(3)
    plsc.subcore_barrier()
    pltpu.sync_copy(acc.at[pl.ds(lo, RPT)], out_hbm.at[c_ax, pl.ds(lo, RPT)])


def _make_edge_pass(H):
    mesh = plsc.VectorSubcoreMesh(core_axis_name="c", subcore_axis_name="s")
    return pl.kernel(
        functools.partial(_edge_body, H),
        mesh=mesh,
        compiler_params=pltpu.CompilerParams(use_tc_tiling_on_sc=False),
        out_type=jax.ShapeDtypeStruct((NC, NP, 16), jnp.float32),
        scratch_types=(
            [pltpu.VMEM((CHUNK,), jnp.int32)] * (5 * NB)
            + [pltpu.VMEM((CHUNK, 16), jnp.float32)] * (3 * NB)
            + [pltpu.VMEM_SHARED((NP, 16), jnp.float32)]
            + [pltpu.SemaphoreType.DMA] * (3 * NB)
        ),
    )


_edge_pass = {4: _make_edge_pass(4), 1: _make_edge_pass(1)}


def _leaky(t):
    return jnp.maximum(t, 0.2 * t)


def _gat_layer(x, srcp, dstp, W, att_src, att_dst, heads, out_ch):
    h = x @ W
    hr = h.reshape(N, heads, out_ch)
    a_src = (hr * att_src).sum(-1)
    a_dst = (hr * att_dst).sum(-1)

    one = jnp.ones((N, 1), jnp.float32)
    z4 = jnp.zeros((N, 4), jnp.float32)
    z10 = jnp.zeros((N, 10), jnp.float32)
    if heads == 4:
        rows = [jnp.concatenate(
            [h[:, 8 * c:8 * c + 8], one, one,
             a_src[:, 2 * c:2 * c + 1], a_src[:, 2 * c + 1:2 * c + 2], z4], 1)
            for c in range(2)]
        drows = [jnp.concatenate(
            [z10, a_dst[:, 2 * c:2 * c + 1], a_dst[:, 2 * c + 1:2 * c + 2],
             z4], 1) for c in range(2)]
    else:
        rows = [jnp.concatenate(
            [h[:, 8 * c:8 * c + 8], one, one, a_src, a_src, z4], 1)
            for c in range(2)]
        drows = [jnp.concatenate([z10, a_dst, a_dst, z4], 1)
                 for _ in range(2)]

    sentinel = jnp.zeros((8, 16), jnp.float32)
    stab = jnp.concatenate([rows[0], rows[1], sentinel], 0)
    dtab = jnp.concatenate([drows[0], drows[1], sentinel], 0)

    acc = _edge_pass[heads](srcp, dstp, stab, dtab)
    ws = jnp.exp(_leaky(a_src + a_dst))
    num = jnp.concatenate([acc[0, :N, 0:8], acc[1, :N, 0:8]], 1)
    num = num + (hr * ws[..., None]).reshape(N, 16)
    if heads == 4:
        den = jnp.concatenate([acc[0, :N, 8:10], acc[1, :N, 8:10]], 1) + ws
    else:
        den = acc[0, :N, 8:9] + ws
    return num / (jnp.repeat(den, out_ch, axis=1) + 1e-16)


def kernel(x, edge_index, W1, att_src1, att_dst1, b1, W2, att_src2, att_dst2,
           b2):
    src = edge_index[0].astype(jnp.int32)
    dst = edge_index[1].astype(jnp.int32)
    pad = EP - E
    srcp = jnp.concatenate([src, jnp.full((pad,), SENT, jnp.int32)])
    dstp = jnp.concatenate([dst, jnp.zeros((pad,), jnp.int32)])

    g1 = _gat_layer(x, srcp, dstp, W1, att_src1, att_dst1, 4, 4) + b1
    x2 = jax.nn.elu(g1)
    g2 = _gat_layer(x2, srcp, dstp, W2, att_src2, att_dst2, 1, 16) + b2
    return g2
